# baseline (speedup 1.0000x reference)
import functools

import jax
import jax.numpy as jnp
from jax import lax
from jax.experimental import pallas as pl
from jax.experimental.pallas import tpu as pltpu
from jax.experimental.pallas import tpu_sc as plsc

SEQ = 200
EMBED_DIM = 64
BATCH = 4096

_info = plsc.get_sparse_core_info()
_NC, _NS = _info.num_cores, _info.num_subcores
_NW = _NC * _NS
_B_PER_W = 8

_sc_mesh = plsc.VectorSubcoreMesh(core_axis_name="c", subcore_axis_name="s")


@functools.partial(
    pl.kernel,
    mesh=_sc_mesh,
    out_type=jax.ShapeDtypeStruct((SEQ, 2), jnp.float32),
    scratch_types=[
        pltpu.VMEM((_B_PER_W, 128), jnp.int32),
        pltpu.VMEM((2, EMBED_DIM), jnp.float32),
        pltpu.VMEM((16,), jnp.float32),
        pltpu.VMEM((_B_PER_W, EMBED_DIM, 128), jnp.float32),
        pltpu.VMEM((_B_PER_W, 2), jnp.float32),
        pltpu.SemaphoreType.DMA,
        pltpu.SemaphoreType.DMA,
    ],
    compiler_params=pltpu.CompilerParams(
        use_tc_tiling_on_sc=True, needs_layout_passes=False
    ),
)
def _sc_embed_linear(
    sent_t_hbm,
    table_t_hbm,
    w_hbm,
    b_hbm,
    out_hbm,
    sent_v,
    w_v,
    b_v,
    stripes_v,
    y_v,
    sem,
    sem2,
):
    wid = lax.axis_index("s") * _NC + lax.axis_index("c")
    base = wid * _B_PER_W

    @pl.when(base < SEQ)
    def _():
        col_blk = (BATCH // 128 - 1) * 128
        s_cp = pltpu.async_copy(
            sent_t_hbm.at[pl.ds(base, _B_PER_W), pl.ds(col_blk, 128)], sent_v, sem2
        )
        w_cp = pltpu.async_copy(w_hbm, w_v, sem2)
        b_cp = pltpu.async_copy(b_hbm, b_v.at[pl.ds(0, 2)], sem2)
        lane = lax.iota(jnp.int32, 16)
        row_mask = lane < _B_PER_W
        s_cp.wait()
        idx_vec = plsc.load_gather(
            sent_v, [lane, jnp.full((16,), 127, jnp.int32)], mask=row_mask
        )
        copies = []
        for j in range(_B_PER_W):
            col0 = pl.multiple_of((idx_vec[j] // 128) * 128, 128)
            copies.append(
                pltpu.async_copy(
                    table_t_hbm.at[:, pl.ds(col0, 128)], stripes_v.at[j], sem
                )
            )
        w_cp.wait()
        b_cp.wait()
        for cp in copies:
            cp.wait()
        col_vec = idx_vec % 128
        zeros = jnp.zeros((16,), jnp.float32)
        zeros_i = jnp.zeros((16,), jnp.int32)
        ones_i = zeros_i + 1
        b_vec = b_v[...]

        def body(d, carry):
            a0, a1 = carry
            dvec = jnp.full((16,), d, jnp.int32)
            vals = plsc.load_gather(
                stripes_v, [lane, dvec, col_vec], mask=row_mask
            )
            w0v = plsc.load_gather(w_v, [zeros_i, dvec])
            w1v = plsc.load_gather(w_v, [ones_i, dvec])
            return a0 + vals * w0v, a1 + vals * w1v

        acc0, acc1 = lax.fori_loop(0, EMBED_DIM, body, (zeros, zeros))
        acc0 = acc0 + b_vec[0]
        acc1 = acc1 + b_vec[1]
        col0i = jnp.zeros((16,), jnp.int32)
        plsc.store_scatter(y_v, [lane, col0i], acc0, mask=row_mask)
        plsc.store_scatter(y_v, [lane, col0i + 1], acc1, mask=row_mask)
        pltpu.sync_copy(y_v, out_hbm.at[pl.ds(base, _B_PER_W)])


def kernel(sentence, emb_table, W, b):
    return _sc_embed_linear(sentence.T.astype(jnp.int32), emb_table.T, W, b)
